# baseline (speedup 1.0000x reference)
import functools

import jax
import jax.numpy as jnp
from jax import lax
from jax.experimental import pallas as pl
from jax.experimental.pallas import tpu as pltpu
from jax.experimental.pallas import tpu_sc as plsc

N = 10000
E = 320000
D = 128
R = 4
NC = 2
NS = 16
NW = NC * NS
EPW = E // NW
CH = 125
CK = 80
NCHUNK = 4
CW = D // NCHUNK
NPAD = 10240
SEGPAD = R * NPAD
TROWS = SEGPAD // NS
NBUF = 3

_f32 = jnp.float32
_bf16 = jnp.bfloat16
_i32 = jnp.int32


def _sc_body(with_count, *refs):
    if with_count:
        (table_hbm, src_hbm, dst_hbm, et_hbm, ones_hbm, zeros_hbm,
         s_out, cnt_out, acc, tbl, srcv, segv, r0, r1, r2,
         g0, g1, g2, s0, s1, s2) = refs
    else:
        (table_hbm, src_hbm, dst_hbm, et_hbm, ones_hbm, zeros_hbm,
         s_out, acc, tbl, srcv, segv, r0, r1, r2,
         g0, g1, g2, s0, s1, s2) = refs
    rows = (r0, r1, r2)
    gsem = (g0, g1, g2)
    ssem = (s0, s1, s2)
    cid = lax.axis_index("c")
    sid = lax.axis_index("s")
    w = cid * NS + sid
    lo = sid * TROWS

    pltpu.sync_copy(dst_hbm.at[w], segv)
    pltpu.sync_copy(et_hbm.at[w], srcv)

    @pl.loop(0, CH)
    def _(i):
        for j in range(CK // 16):
            sl = pl.ds(j * 16, 16)
            segv[i, sl] = srcv[i, sl] * NPAD + segv[i, sl]

    pltpu.sync_copy(src_hbm.at[w], srcv)

    def zero_my_slice():
        pltpu.sync_copy(zeros_hbm, acc.at[pl.ds(lo, TROWS)])

    def gstart(i, j, table=None):
        pltpu.async_copy(tbl.at[srcv.at[i]], rows[j], gsem[j])

    def gwait(j, table=None):
        pltpu.make_async_copy(tbl.at[srcv.at[0]], rows[j], gsem[j]).wait()

    def sstart(i, j):
        pltpu.async_copy(rows[j], acc.at[segv.at[i]], ssem[j], add=True)

    def swait(j):
        pltpu.make_async_copy(rows[j], acc.at[segv.at[0]], ssem[j]).wait()

    trows = N // NS
    for c in range(NCHUNK):
        zero_my_slice()
        pltpu.sync_copy(
            table_hbm.at[pl.ds(sid * trows, trows), pl.ds(c * CW, CW)],
            tbl.at[pl.ds(sid * trows, trows)])
        plsc.subcore_barrier()
        table = None

        gstart(0, 0, table)
        for i in range(NBUF - 1):
            gwait(i % NBUF, table)
            sstart(i, i % NBUF)
            gstart(i + 1, (i + 1) % NBUF, table)

        @pl.loop(0, (CH - NBUF) // NBUF)
        def _(k):
            base = NBUF - 1 + k * NBUF
            for j in range(NBUF):
                i = base + j
                jj = (NBUF - 1 + j) % NBUF
                jn = (jj + 1) % NBUF
                gwait(jj, table)
                sstart(i, jj)
                swait(jn)
                gstart(i + 1, jn, table)

        tail = NBUF - 1 + ((CH - NBUF) // NBUF) * NBUF
        for i in range(tail, CH):
            jj = i % NBUF
            gwait(jj, table)
            if i + 1 < CH:
                jn = (i + 1) % NBUF
                swait(jn)
                gstart(i + 1, jn, table)
            sstart(i, jj)
        for j in range(NBUF):
            swait(j)

        plsc.subcore_barrier()
        pltpu.sync_copy(acc.at[pl.ds(lo, TROWS)],
                        s_out.at[cid, pl.ds(lo, TROWS), pl.ds(c * CW, CW)])

    if with_count:
        zero_my_slice()
        pltpu.sync_copy(ones_hbm, r0)
        plsc.subcore_barrier()

        def cstart(i):
            pltpu.async_copy(r0, acc.at[segv.at[i]], ssem[0], add=True)

        def cwait():
            pltpu.make_async_copy(r0, acc.at[segv.at[0]], ssem[0]).wait()

        for i in range(NBUF):
            cstart(i)

        @pl.loop(NBUF, CH)
        def _(i):
            cwait()
            cstart(i)

        for _ in range(NBUF):
            cwait()

        plsc.subcore_barrier()
        pltpu.sync_copy(acc.at[pl.ds(lo, TROWS), pl.ds(0, 16)],
                        cnt_out.at[cid, pl.ds(lo, TROWS)])


def _sc_segment_sums(table, srcT, dstT, etT, ones_rows, zeros_rows,
                     with_count):
    out_type = [jax.ShapeDtypeStruct((NC, SEGPAD, D), _bf16)]
    if with_count:
        out_type.append(jax.ShapeDtypeStruct((NC, SEGPAD, 16), _bf16))
    mesh = plsc.VectorSubcoreMesh(core_axis_name="c", subcore_axis_name="s",
                                  num_cores=NC, num_subcores=NS)
    fn = pl.kernel(
        functools.partial(_sc_body, with_count),
        out_type=tuple(out_type),
        mesh=mesh,
        scratch_types=[
            pltpu.VMEM_SHARED((SEGPAD, CW), _bf16),
            pltpu.VMEM_SHARED((N, CW), _bf16),
            pltpu.VMEM((CH, CK), _i32),
            pltpu.VMEM((CH, CK), _i32),
            pltpu.VMEM((CK, CW), _bf16),
            pltpu.VMEM((CK, CW), _bf16),
            pltpu.VMEM((CK, CW), _bf16),
            pltpu.SemaphoreType.DMA,
            pltpu.SemaphoreType.DMA,
            pltpu.SemaphoreType.DMA,
            pltpu.SemaphoreType.DMA,
            pltpu.SemaphoreType.DMA,
            pltpu.SemaphoreType.DMA,
        ],
        compiler_params=pltpu.CompilerParams(use_tc_tiling_on_sc=False),
    )
    return fn(table, srcT, dstT, etT, ones_rows, zeros_rows)


def _dense_body(relu, x_ref, s_ref, cnt_ref, root_ref, b_ref,
                w_ref, out_ref):
    x = x_ref[...]
    acc = jnp.dot(x, root_ref[...], preferred_element_type=_f32) + b_ref[...]
    s = s_ref[...].astype(_f32)
    ssum = s[0] + s[1]
    cnt = cnt_ref[...].astype(_f32)
    cd = jnp.maximum(cnt[0, :, :, 0] + cnt[1, :, :, 0], 1.0)
    w = w_ref[...]
    for r in range(R):
        sr = ssum[r] / cd[r][:, None]
        acc = acc + jnp.dot(sr, w[r], preferred_element_type=_f32)
    if relu:
        acc = jnp.maximum(acc, 0.0)
    out_ref[...] = acc


def _dense_layer(x, s5, cnt3, root, b, w, relu, nb=2048):
    grid = (pl.cdiv(N, nb),)
    out_shape = jax.ShapeDtypeStruct((N, D), _f32)
    out_specs = pl.BlockSpec((nb, D), lambda i: (i, 0))
    return pl.pallas_call(
        functools.partial(_dense_body, relu),
        grid=grid,
        in_specs=[
            pl.BlockSpec((nb, D), lambda i: (i, 0)),
            pl.BlockSpec((NC, R, nb, D), lambda i: (0, 0, i, 0)),
            pl.BlockSpec((NC, R, nb, 16), lambda i: (0, 0, i, 0)),
            pl.BlockSpec((D, D), lambda i: (0, 0)),
            pl.BlockSpec((1, D), lambda i: (0, 0)),
            pl.BlockSpec((R, D, D), lambda i: (0, 0, 0)),
        ],
        out_specs=out_specs,
        out_shape=out_shape,
    )(x, s5, cnt3, root, b, w)


def kernel(classic_features, edge_index, edge_type, W1, root1, b1, W2,
           root2, b2):
    x = classic_features

    src = edge_index[0].reshape(NW, CH, CK)
    dst = edge_index[1].reshape(NW, CH, CK)
    et = edge_type.reshape(NW, CH, CK)
    ones_rows = jnp.ones((CK, CW), _bf16)
    zeros_rows = jnp.zeros((TROWS, CW), _bf16)

    s1, cnt = _sc_segment_sums(x.astype(_bf16), src, dst, et, ones_rows,
                               zeros_rows, with_count=True)
    s1 = s1.reshape(NC, R, NPAD, D)
    cnt3 = cnt.reshape(NC, R, NPAD, 16)

    h = _dense_layer(x, s1, cnt3, root1, b1.reshape(1, D), W1, relu=True)

    (s2,) = _sc_segment_sums(h.astype(_bf16), src, dst, et, ones_rows,
                             zeros_rows, with_count=False)
    s2 = s2.reshape(NC, R, NPAD, D)

    out = _dense_layer(h, s2, cnt3, root2, b2.reshape(1, D), W2, relu=False)
    return out


# speedup vs baseline: 1.1761x; 1.1761x over previous
import functools

import jax
import jax.numpy as jnp
from jax import lax
from jax.experimental import pallas as pl
from jax.experimental.pallas import tpu as pltpu
from jax.experimental.pallas import tpu_sc as plsc

N = 10000
E = 320000
D = 128
R = 4
NC = 2
NS = 16
NW = NC * NS
EPW = E // NW
CH = 125
CK = 80
NCHUNK = 2
CW = D // NCHUNK
NPAD = 10240
SEGPAD = R * NPAD
TROWS = SEGPAD // NS
NBUF = 3

_f32 = jnp.float32
_bf16 = jnp.bfloat16
_i32 = jnp.int32


def _sc_body(with_count, *refs):
    if with_count:
        (table_hbm, src_hbm, dst_hbm, et_hbm, ones_hbm, zeros_hbm,
         s_out, cnt_out, acc, tbl, srcv, segv, r0, r1, r2,
         g0, g1, g2, s0, s1, s2) = refs
    else:
        (table_hbm, src_hbm, dst_hbm, et_hbm, ones_hbm, zeros_hbm,
         s_out, acc, tbl, srcv, segv, r0, r1, r2,
         g0, g1, g2, s0, s1, s2) = refs
    rows = (r0, r1, r2)
    gsem = (g0, g1, g2)
    ssem = (s0, s1, s2)
    cid = lax.axis_index("c")
    sid = lax.axis_index("s")
    w = cid * NS + sid
    lo = sid * TROWS

    pltpu.sync_copy(dst_hbm.at[w], segv)
    pltpu.sync_copy(et_hbm.at[w], srcv)

    @pl.loop(0, CH)
    def _(i):
        for j in range(CK // 16):
            sl = pl.ds(j * 16, 16)
            segv[i, sl] = srcv[i, sl] * NPAD + segv[i, sl]

    pltpu.sync_copy(src_hbm.at[w], srcv)

    def zero_my_slice():
        pltpu.sync_copy(zeros_hbm, acc.at[pl.ds(lo, TROWS)])

    def gstart(i, j, table=None):
        pltpu.async_copy(tbl.at[srcv.at[i]], rows[j], gsem[j])

    def gwait(j, table=None):
        pltpu.make_async_copy(tbl.at[srcv.at[0]], rows[j], gsem[j]).wait()

    def sstart(i, j):
        pltpu.async_copy(rows[j], acc.at[segv.at[i]], ssem[j], add=True)

    def swait(j):
        pltpu.make_async_copy(rows[j], acc.at[segv.at[0]], ssem[j]).wait()

    trows = N // NS
    for c in range(NCHUNK):
        zero_my_slice()
        pltpu.sync_copy(
            table_hbm.at[pl.ds(sid * trows, trows), pl.ds(c * CW, CW)],
            tbl.at[pl.ds(sid * trows, trows)])
        plsc.subcore_barrier()
        table = None

        gstart(0, 0, table)
        for i in range(NBUF - 1):
            gwait(i % NBUF, table)
            sstart(i, i % NBUF)
            gstart(i + 1, (i + 1) % NBUF, table)

        @pl.loop(0, (CH - NBUF) // NBUF)
        def _(k):
            base = NBUF - 1 + k * NBUF
            for j in range(NBUF):
                i = base + j
                jj = (NBUF - 1 + j) % NBUF
                jn = (jj + 1) % NBUF
                gwait(jj, table)
                sstart(i, jj)
                swait(jn)
                gstart(i + 1, jn, table)

        tail = NBUF - 1 + ((CH - NBUF) // NBUF) * NBUF
        for i in range(tail, CH):
            jj = i % NBUF
            gwait(jj, table)
            if i + 1 < CH:
                jn = (i + 1) % NBUF
                swait(jn)
                gstart(i + 1, jn, table)
            sstart(i, jj)
        for j in range(NBUF):
            swait(j)

        plsc.subcore_barrier()
        pltpu.sync_copy(acc.at[pl.ds(lo, TROWS)],
                        s_out.at[cid, pl.ds(lo, TROWS), pl.ds(c * CW, CW)])

    if with_count:
        zero_my_slice()
        pltpu.sync_copy(ones_hbm, r0)
        plsc.subcore_barrier()

        def cstart(i):
            pltpu.async_copy(r0, acc.at[segv.at[i]], ssem[0], add=True)

        def cwait():
            pltpu.make_async_copy(r0, acc.at[segv.at[0]], ssem[0]).wait()

        for i in range(NBUF):
            cstart(i)

        @pl.loop(NBUF, CH)
        def _(i):
            cwait()
            cstart(i)

        for _ in range(NBUF):
            cwait()

        plsc.subcore_barrier()
        pltpu.sync_copy(acc.at[pl.ds(lo, TROWS), pl.ds(0, 16)],
                        cnt_out.at[cid, pl.ds(lo, TROWS)])


def _sc_segment_sums(table, srcT, dstT, etT, ones_rows, zeros_rows,
                     with_count):
    out_type = [jax.ShapeDtypeStruct((NC, SEGPAD, D), _bf16)]
    if with_count:
        out_type.append(jax.ShapeDtypeStruct((NC, SEGPAD, 16), _bf16))
    mesh = plsc.VectorSubcoreMesh(core_axis_name="c", subcore_axis_name="s",
                                  num_cores=NC, num_subcores=NS)
    fn = pl.kernel(
        functools.partial(_sc_body, with_count),
        out_type=tuple(out_type),
        mesh=mesh,
        scratch_types=[
            pltpu.VMEM_SHARED((SEGPAD, CW), _bf16),
            pltpu.VMEM_SHARED((N, CW), _bf16),
            pltpu.VMEM((CH, CK), _i32),
            pltpu.VMEM((CH, CK), _i32),
            pltpu.VMEM((CK, CW), _bf16),
            pltpu.VMEM((CK, CW), _bf16),
            pltpu.VMEM((CK, CW), _bf16),
            pltpu.SemaphoreType.DMA,
            pltpu.SemaphoreType.DMA,
            pltpu.SemaphoreType.DMA,
            pltpu.SemaphoreType.DMA,
            pltpu.SemaphoreType.DMA,
            pltpu.SemaphoreType.DMA,
        ],
        compiler_params=pltpu.CompilerParams(use_tc_tiling_on_sc=False),
    )
    return fn(table, srcT, dstT, etT, ones_rows, zeros_rows)


def _dense_body(relu, x_ref, s_ref, cnt_ref, root_ref, b_ref,
                w_ref, out_ref):
    x = x_ref[...]
    acc = jnp.dot(x, root_ref[...], preferred_element_type=_f32) + b_ref[...]
    s = s_ref[...].astype(_f32)
    ssum = s[0] + s[1]
    cnt = cnt_ref[...].astype(_f32)
    cd = jnp.maximum(cnt[0, :, :, 0] + cnt[1, :, :, 0], 1.0)
    w = w_ref[...]
    for r in range(R):
        sr = ssum[r] / cd[r][:, None]
        acc = acc + jnp.dot(sr, w[r], preferred_element_type=_f32)
    if relu:
        acc = jnp.maximum(acc, 0.0)
    out_ref[...] = acc


def _dense_layer(x, s5, cnt3, root, b, w, relu, nb=2048):
    grid = (pl.cdiv(N, nb),)
    out_shape = jax.ShapeDtypeStruct((N, D), _f32)
    out_specs = pl.BlockSpec((nb, D), lambda i: (i, 0))
    return pl.pallas_call(
        functools.partial(_dense_body, relu),
        grid=grid,
        in_specs=[
            pl.BlockSpec((nb, D), lambda i: (i, 0)),
            pl.BlockSpec((NC, R, nb, D), lambda i: (0, 0, i, 0)),
            pl.BlockSpec((NC, R, nb, 16), lambda i: (0, 0, i, 0)),
            pl.BlockSpec((D, D), lambda i: (0, 0)),
            pl.BlockSpec((1, D), lambda i: (0, 0)),
            pl.BlockSpec((R, D, D), lambda i: (0, 0, 0)),
        ],
        out_specs=out_specs,
        out_shape=out_shape,
    )(x, s5, cnt3, root, b, w)


def kernel(classic_features, edge_index, edge_type, W1, root1, b1, W2,
           root2, b2):
    x = classic_features

    src = edge_index[0].reshape(NW, CH, CK)
    dst = edge_index[1].reshape(NW, CH, CK)
    et = edge_type.reshape(NW, CH, CK)
    ones_rows = jnp.ones((CK, CW), _bf16)
    zeros_rows = jnp.zeros((TROWS, CW), _bf16)

    s1, cnt = _sc_segment_sums(x.astype(_bf16), src, dst, et, ones_rows,
                               zeros_rows, with_count=True)
    s1 = s1.reshape(NC, R, NPAD, D)
    cnt3 = cnt.reshape(NC, R, NPAD, 16)

    h = _dense_layer(x, s1, cnt3, root1, b1.reshape(1, D), W1, relu=True)

    (s2,) = _sc_segment_sums(h.astype(_bf16), src, dst, et, ones_rows,
                             zeros_rows, with_count=False)
    s2 = s2.reshape(NC, R, NPAD, D)

    out = _dense_layer(h, s2, cnt3, root2, b2.reshape(1, D), W2, relu=False)
    return out
